# baseline (speedup 1.0000x reference)
import functools

import jax
import jax.numpy as jnp
from jax import lax
from jax.experimental import pallas as pl
from jax.experimental.pallas import tpu as pltpu


def _moe_kernel(flat_ref, logits_ref, mask_ref, w_ref, b_ref,
                out_ref, gl_ref, acc_ref, gcol_ref, fbf_ref,
                *, n_experts, seg_len):
    e = pl.program_id(0)
    n_rows = flat_ref.shape[0]
    n_b = logits_ref.shape[0]

    @pl.when(e == 0)
    def _prologue():
        fbf_ref[...] = flat_ref[...].astype(jnp.bfloat16)

        lg = logits_ref[...]
        m = (mask_ref[...] == 1).astype(jnp.float32)
        p = lg - jnp.max(lg, axis=1, keepdims=True)
        p = jnp.exp(p)
        p = p / jnp.sum(p, axis=1, keepdims=True)
        g = p * m
        g = g / (jnp.sum(g, axis=1, keepdims=True) + 1e-9)

        rowid = lax.broadcasted_iota(jnp.int32, (n_rows, n_b), 0) // seg_len
        colid = lax.broadcasted_iota(jnp.int32, (n_rows, n_b), 1)
        onehot = (rowid == colid).astype(jnp.float32)
        gcol_ref[...] = jnp.dot(onehot, g, preferred_element_type=jnp.float32)

        gb = jnp.dot(g, b_ref[...], preferred_element_type=jnp.float32)
        acc_ref[...] = jnp.dot(onehot, gb, preferred_element_type=jnp.float32)

        s = jnp.sum(p * m) / n_b
        gl_ref[...] = ((1.0 - s) * (1.0 - s)).reshape(1, 1)

    mm = jnp.dot(fbf_ref[...], w_ref[0].astype(jnp.bfloat16),
                 preferred_element_type=jnp.float32)
    lane = lax.broadcasted_iota(jnp.int32, gcol_ref.shape, 1)
    ge = jnp.sum(gcol_ref[...] * (lane == e).astype(jnp.float32),
                 axis=1, keepdims=True)
    acc_ref[...] = acc_ref[...] + ge * mm

    @pl.when(e == n_experts - 1)
    def _finish():
        out_ref[...] = acc_ref[...].astype(jnp.bfloat16)


@jax.jit
def kernel(cycle_curve_data, logits, moe_masks, expert_w, expert_b):
    B, L, C, CLEN = cycle_curve_data.shape
    E, F, DM = expert_w.shape
    N = B * L
    flat = cycle_curve_data.reshape(N, C * CLEN)
    masks = moe_masks.astype(jnp.int32)

    out, gl = pl.pallas_call(
        functools.partial(_moe_kernel, n_experts=E, seg_len=L),
        grid=(E,),
        in_specs=[
            pl.BlockSpec((N, F), lambda e: (0, 0)),
            pl.BlockSpec((B, E), lambda e: (0, 0)),
            pl.BlockSpec((B, E), lambda e: (0, 0)),
            pl.BlockSpec((1, F, DM), lambda e: (e, 0, 0)),
            pl.BlockSpec((E, DM), lambda e: (0, 0)),
        ],
        out_specs=[
            pl.BlockSpec((N, DM), lambda e: (0, 0)),
            pl.BlockSpec((1, 1), lambda e: (0, 0)),
        ],
        out_shape=[
            jax.ShapeDtypeStruct((N, DM), jnp.bfloat16),
            jax.ShapeDtypeStruct((1, 1), jnp.float32),
        ],
        scratch_shapes=[
            pltpu.VMEM((N, DM), jnp.float32),
            pltpu.VMEM((N, E), jnp.float32),
            pltpu.VMEM((N, F), jnp.bfloat16),
        ],
    )(flat, logits, masks, expert_w, expert_b)

    return out.reshape(B, L, DM), gl.reshape(())


# speedup vs baseline: 1.0049x; 1.0049x over previous
import functools

import jax
import jax.numpy as jnp
from jax import lax
from jax.experimental import pallas as pl
from jax.experimental.pallas import tpu as pltpu


def _moe_kernel(flat_ref, logits_ref, mask_ref, w_ref, b_ref,
                out_ref, gl_ref, acc_ref, gcol_ref, *, n_experts, seg_len):
    e = pl.program_id(0)
    n_rows = flat_ref.shape[0]
    n_b = logits_ref.shape[0]

    @pl.when(e == 0)
    def _prologue():
        lg = logits_ref[...]
        m = (mask_ref[...] == 1).astype(jnp.float32)
        p = lg - jnp.max(lg, axis=1, keepdims=True)
        p = jnp.exp(p)
        p = p / jnp.sum(p, axis=1, keepdims=True)
        g = p * m
        g = g / (jnp.sum(g, axis=1, keepdims=True) + 1e-9)

        rowid = lax.broadcasted_iota(jnp.int32, (n_rows, n_b), 0) // seg_len
        colid = lax.broadcasted_iota(jnp.int32, (n_rows, n_b), 1)
        onehot = (rowid == colid).astype(jnp.float32)
        gcol_ref[...] = jnp.dot(onehot, g, preferred_element_type=jnp.float32)

        gb = jnp.dot(g, b_ref[...], preferred_element_type=jnp.float32)
        acc_ref[...] = jnp.dot(onehot, gb, preferred_element_type=jnp.float32)

        s = jnp.sum(p * m) / n_b
        gl_ref[...] = ((1.0 - s) * (1.0 - s)).reshape(1, 1)

    mm = jnp.dot(flat_ref[...], w_ref[0],
                 preferred_element_type=jnp.float32)
    lane = lax.broadcasted_iota(jnp.int32, gcol_ref.shape, 1)
    ge = jnp.sum(gcol_ref[...] * (lane == e).astype(jnp.float32),
                 axis=1, keepdims=True)
    acc_ref[...] = acc_ref[...] + ge * mm

    @pl.when(e == n_experts - 1)
    def _finish():
        out_ref[...] = acc_ref[...].astype(jnp.bfloat16)


@jax.jit
def kernel(cycle_curve_data, logits, moe_masks, expert_w, expert_b):
    B, L, C, CLEN = cycle_curve_data.shape
    E, F, DM = expert_w.shape
    N = B * L
    flat = cycle_curve_data.reshape(N, C * CLEN)
    masks = moe_masks.astype(jnp.int32)

    out, gl = pl.pallas_call(
        functools.partial(_moe_kernel, n_experts=E, seg_len=L),
        grid=(E,),
        in_specs=[
            pl.BlockSpec((N, F), lambda e: (0, 0)),
            pl.BlockSpec((B, E), lambda e: (0, 0)),
            pl.BlockSpec((B, E), lambda e: (0, 0)),
            pl.BlockSpec((1, F, DM), lambda e: (e, 0, 0)),
            pl.BlockSpec((E, DM), lambda e: (0, 0)),
        ],
        out_specs=[
            pl.BlockSpec((N, DM), lambda e: (0, 0)),
            pl.BlockSpec((1, 1), lambda e: (0, 0)),
        ],
        out_shape=[
            jax.ShapeDtypeStruct((N, DM), jnp.bfloat16),
            jax.ShapeDtypeStruct((1, 1), jnp.float32),
        ],
        scratch_shapes=[
            pltpu.VMEM((N, DM), jnp.float32),
            pltpu.VMEM((N, E), jnp.float32),
        ],
    )(flat, logits, masks, expert_w, expert_b)

    return out.reshape(B, L, DM), gl.reshape(())


# speedup vs baseline: 1.0114x; 1.0065x over previous
import functools

import jax
import jax.numpy as jnp
from jax import lax
from jax.experimental import pallas as pl
from jax.experimental.pallas import tpu as pltpu


def _moe_kernel(flat_ref, logits_ref, mask_ref, w_ref, b_ref,
                out_ref, gl_ref, acc_ref, gcol_ref, *, n_experts, seg_len):
    e = pl.program_id(0)
    n_rows = flat_ref.shape[0]
    n_b = logits_ref.shape[0]

    @pl.when(e == 0)
    def _prologue():
        lg = logits_ref[...]
        m = (mask_ref[...] == 1).astype(jnp.float32)
        p = lg - jnp.max(lg, axis=1, keepdims=True)
        p = jnp.exp(p)
        p = p / jnp.sum(p, axis=1, keepdims=True)
        g = p * m
        g = g / (jnp.sum(g, axis=1, keepdims=True) + 1e-9)

        rowid = lax.broadcasted_iota(jnp.int32, (n_rows, n_b), 0) // seg_len
        colid = lax.broadcasted_iota(jnp.int32, (n_rows, n_b), 1)
        onehot = (rowid == colid).astype(jnp.float32)
        gcol_ref[...] = jnp.dot(onehot, g, preferred_element_type=jnp.float32)

        gb = jnp.dot(g, b_ref[...], preferred_element_type=jnp.float32)
        acc_ref[...] = jnp.dot(onehot, gb, preferred_element_type=jnp.float32)

        s = jnp.sum(p * m) / n_b
        gl_ref[...] = ((1.0 - s) * (1.0 - s)).reshape(1, 1)

    mm = jnp.dot(flat_ref[...], w_ref[0],
                 preferred_element_type=jnp.float32)
    lane = lax.broadcasted_iota(jnp.int32, gcol_ref.shape, 1)
    ge = jnp.sum(gcol_ref[...] * (lane == e).astype(jnp.float32),
                 axis=1, keepdims=True)
    acc_ref[...] = acc_ref[...] + ge * mm

    @pl.when(e == n_experts - 1)
    def _finish():
        out_ref[...] = acc_ref[...].astype(jnp.bfloat16)


@jax.jit
def kernel(cycle_curve_data, logits, moe_masks, expert_w, expert_b):
    B, L, C, CLEN = cycle_curve_data.shape
    E, F, DM = expert_w.shape
    N = B * L
    flat = cycle_curve_data.reshape(N, C * CLEN).astype(jnp.bfloat16)
    masks = moe_masks.astype(jnp.int32)

    out, gl = pl.pallas_call(
        functools.partial(_moe_kernel, n_experts=E, seg_len=L),
        grid=(E,),
        in_specs=[
            pl.BlockSpec((N, F), lambda e: (0, 0)),
            pl.BlockSpec((B, E), lambda e: (0, 0)),
            pl.BlockSpec((B, E), lambda e: (0, 0)),
            pl.BlockSpec((1, F, DM), lambda e: (e, 0, 0)),
            pl.BlockSpec((E, DM), lambda e: (0, 0)),
        ],
        out_specs=[
            pl.BlockSpec((N, DM), lambda e: (0, 0)),
            pl.BlockSpec((1, 1), lambda e: (0, 0)),
        ],
        out_shape=[
            jax.ShapeDtypeStruct((N, DM), jnp.bfloat16),
            jax.ShapeDtypeStruct((1, 1), jnp.float32),
        ],
        scratch_shapes=[
            pltpu.VMEM((N, DM), jnp.float32),
            pltpu.VMEM((N, E), jnp.float32),
        ],
    )(flat, logits, masks, expert_w.astype(jnp.bfloat16), expert_b)

    return out.reshape(B, L, DM), gl.reshape(())


# speedup vs baseline: 1.5606x; 1.5430x over previous
import functools

import jax
import jax.numpy as jnp
from jax import lax
from jax.experimental import pallas as pl
from jax.experimental.pallas import tpu as pltpu


def _moe_kernel(flat_ref, logits_ref, mask_ref, w_ref, b_ref,
                out_ref, gl_ref, acc_ref, gcol_ref, *, n_experts, seg_len):
    e = pl.program_id(0)
    n_rows = flat_ref.shape[0]
    n_b = logits_ref.shape[0]

    @pl.when(e == 0)
    def _prologue():
        lg = logits_ref[...]
        m = (mask_ref[...] == 1).astype(jnp.float32)
        p = lg - jnp.max(lg, axis=1, keepdims=True)
        p = jnp.exp(p)
        p = p / jnp.sum(p, axis=1, keepdims=True)
        g = p * m
        g = g / (jnp.sum(g, axis=1, keepdims=True) + 1e-9)

        rowid = lax.broadcasted_iota(jnp.int32, (n_rows, n_b), 0) // seg_len
        colid = lax.broadcasted_iota(jnp.int32, (n_rows, n_b), 1)
        onehot = (rowid == colid).astype(jnp.float32)
        gcol_ref[...] = jnp.dot(onehot, g, preferred_element_type=jnp.float32)

        gb = jnp.dot(g, b_ref[...], preferred_element_type=jnp.float32)
        acc_ref[...] = jnp.dot(onehot, gb, preferred_element_type=jnp.float32)

        s = jnp.sum(p * m) / n_b
        gl_ref[...] = ((1.0 - s) * (1.0 - s)).reshape(1, 1)

    mm = jnp.dot(flat_ref[...], w_ref[0].astype(jnp.bfloat16),
                 preferred_element_type=jnp.float32)
    lane = lax.broadcasted_iota(jnp.int32, gcol_ref.shape, 1)
    ge = jnp.sum(gcol_ref[...] * (lane == e).astype(jnp.float32),
                 axis=1, keepdims=True)
    acc_ref[...] = acc_ref[...] + ge * mm

    @pl.when(e == n_experts - 1)
    def _finish():
        out_ref[...] = acc_ref[...].astype(jnp.bfloat16)


@jax.jit
def kernel(cycle_curve_data, logits, moe_masks, expert_w, expert_b):
    B, L, C, CLEN = cycle_curve_data.shape
    E, F, DM = expert_w.shape
    N = B * L
    flat = cycle_curve_data.reshape(N, C * CLEN).astype(jnp.bfloat16)
    masks = moe_masks.astype(jnp.int32)

    out, gl = pl.pallas_call(
        functools.partial(_moe_kernel, n_experts=E, seg_len=L),
        grid=(E,),
        in_specs=[
            pl.BlockSpec((N, F), lambda e: (0, 0)),
            pl.BlockSpec((B, E), lambda e: (0, 0)),
            pl.BlockSpec((B, E), lambda e: (0, 0)),
            pl.BlockSpec((1, F, DM), lambda e: (e, 0, 0)),
            pl.BlockSpec((E, DM), lambda e: (0, 0)),
        ],
        out_specs=[
            pl.BlockSpec((N, DM), lambda e: (0, 0)),
            pl.BlockSpec((1, 1), lambda e: (0, 0)),
        ],
        out_shape=[
            jax.ShapeDtypeStruct((N, DM), jnp.bfloat16),
            jax.ShapeDtypeStruct((1, 1), jnp.float32),
        ],
        scratch_shapes=[
            pltpu.VMEM((N, DM), jnp.float32),
            pltpu.VMEM((N, E), jnp.float32),
        ],
    )(flat, logits, masks, expert_w, expert_b)

    return out.reshape(B, L, DM), gl.reshape(())
